# baseline (speedup 1.0000x reference)
import functools

import jax
import jax.numpy as jnp
from jax import lax
from jax.experimental import pallas as pl
from jax.experimental.pallas import tpu as pltpu
from jax.experimental.pallas import tpu_sc as plsc

N = 10000
E = 320000
D = 128
NC = 2
NS = 16
NW = NC * NS
EPW = E // NW
CHUNK_A = 80
CHUNK_B = 125
RPT = 624
TAIL = N - RPT * NS
ZROWS = 24
CW = 16


def _fill2d(ref, nrows, ncols, value):
    vec = jnp.full((16,), value, jnp.float32)

    def row(r, _):
        def col(c, _2):
            ref[r, pl.ds(c * 16, 16)] = vec
            return 0
        return lax.fori_loop(0, ncols // 16, col, 0)
    lax.fori_loop(0, nrows, row, 0)


def _make_sc_agg(FD, CHUNK, NB):
    NCHUNK = EPW // CHUNK
    with_counts = False
    mesh = plsc.VectorSubcoreMesh(core_axis_name="c", subcore_axis_name="s")
    out_type = [jax.ShapeDtypeStruct((NC, N, FD), jnp.float32)]
    scratch = [
        pltpu.VMEM_SHARED((N, FD), jnp.float32),
        pltpu.VMEM((NCHUNK, CHUNK), jnp.int32),
    ]
    scratch += [pltpu.VMEM((CHUNK,), jnp.int32) for _ in range(NB)]
    scratch += [pltpu.VMEM((CHUNK, FD), jnp.float32) for _ in range(NB)]
    scratch += [pltpu.SemaphoreType.DMA for _ in range(3 * NB)]
    if with_counts:
        out_type.append(jax.ShapeDtypeStruct((NC, N, CW), jnp.float32))
        scratch += [
            pltpu.VMEM_SHARED((N, CW), jnp.float32),
            pltpu.VMEM((CHUNK, CW), jnp.float32),
        ]
        scratch += [pltpu.SemaphoreType.DMA for _ in range(NB)]

    @functools.partial(pl.kernel, mesh=mesh, out_type=tuple(out_type),
                       scratch_types=tuple(scratch),
                       compiler_params=pltpu.CompilerParams(
                           use_tc_tiling_on_sc=False))
    def k(x_hbm, src_hbm, dst_hbm, *refs):
        it = iter(refs)
        agg_out = next(it)
        cnt_out = next(it) if with_counts else None
        agg_sh = next(it)
        didx = next(it)
        six = tuple(next(it) for _ in range(NB))
        rows = tuple(next(it) for _ in range(NB))
        sg = tuple(next(it) for _ in range(NB))
        ss = tuple(next(it) for _ in range(NB))
        si = tuple(next(it) for _ in range(NB))
        if with_counts:
            cnt_sh = next(it)
            ones = next(it)
            sc = tuple(next(it) for _ in range(NB))
        cid = lax.axis_index("c")
        sid = lax.axis_index("s")
        wid = cid * NS + sid
        is_last = sid == NS - 1
        brow = wid * NCHUNK

        pltpu.async_copy(dst_hbm.at[pl.ds(brow, NCHUNK)], didx, si[0])
        pltpu.sync_copy(src_hbm.at[brow], six[0])
        pltpu.async_copy(x_hbm.at[six[0]], rows[0], sg[0])
        pltpu.async_copy(src_hbm.at[brow + 1], six[1], si[1])

        _fill2d(rows[1], CHUNK, FD, 0.0)
        for j in range(RPT // CHUNK):
            pltpu.sync_copy(rows[1], agg_sh.at[pl.ds(sid * RPT + j * CHUNK, CHUNK)])
        rem = RPT - (RPT // CHUNK) * CHUNK
        pltpu.sync_copy(rows[1].at[pl.ds(0, rem)],
                        agg_sh.at[pl.ds(sid * RPT + RPT - rem, rem)])

        @pl.when(is_last)
        def _():
            pltpu.sync_copy(rows[1].at[pl.ds(0, TAIL)],
                            agg_sh.at[pl.ds(NS * RPT, TAIL)])
        if with_counts:
            _fill2d(ones, CHUNK, CW, 0.0)
            for j in range(RPT // CHUNK):
                pltpu.sync_copy(ones, cnt_sh.at[pl.ds(sid * RPT + j * CHUNK, CHUNK)])
            pltpu.sync_copy(ones.at[pl.ds(0, rem)],
                            cnt_sh.at[pl.ds(sid * RPT + RPT - rem, rem)])

            @pl.when(is_last)
            def _():
                pltpu.sync_copy(ones.at[pl.ds(0, TAIL)],
                                cnt_sh.at[pl.ds(NS * RPT, TAIL)])
            _fill2d(ones, CHUNK, CW, 1.0)
        pltpu.make_async_copy(dst_hbm.at[pl.ds(brow, NCHUNK)], didx,
                              si[0]).wait()
        plsc.subcore_barrier()

        def chunk(i, _):
            def piece(b):
                b1 = (b + 1) % NB
                b2 = (b + 2) % NB
                pltpu.make_async_copy(x_hbm.at[six[b]], rows[b], sg[b]).wait()
                nxt = jnp.minimum(i + 2, NCHUNK - 1)
                pltpu.async_copy(src_hbm.at[brow + nxt], six[b2], si[b2])
                pltpu.async_copy(rows[b], agg_sh.at[didx.at[i]], ss[b],
                                 add=True)
                if with_counts:
                    pltpu.async_copy(ones, cnt_sh.at[didx.at[i]], sc[b],
                                     add=True)

                @pl.when(i >= NB - 1)
                def _():
                    pltpu.make_async_copy(rows[b1], agg_sh.at[didx.at[i]],
                                          ss[b1]).wait()
                    if with_counts:
                        pltpu.make_async_copy(ones, cnt_sh.at[didx.at[i]],
                                              sc[b1]).wait()
                pltpu.make_async_copy(src_hbm.at[brow], six[b1], si[b1]).wait()
                pltpu.async_copy(x_hbm.at[six[b1]], rows[b1], sg[b1])

            lax.switch(lax.rem(i, NB),
                       [functools.partial(piece, b) for b in range(NB)])
            return 0
        lax.fori_loop(0, NCHUNK - 1, chunk, 0)

        lbl = (NCHUNK - 1) % NB
        pltpu.make_async_copy(x_hbm.at[six[lbl]], rows[lbl], sg[lbl]).wait()
        pltpu.async_copy(rows[lbl], agg_sh.at[didx.at[NCHUNK - 1]], ss[lbl],
                         add=True)
        if with_counts:
            pltpu.async_copy(ones, cnt_sh.at[didx.at[NCHUNK - 1]], sc[lbl],
                             add=True)
        for kq in range(NB):
            bq = (NCHUNK - 1 - kq) % NB
            pltpu.make_async_copy(rows[bq], agg_sh.at[didx.at[0]],
                                  ss[bq]).wait()
            if with_counts:
                pltpu.make_async_copy(ones, cnt_sh.at[didx.at[0]],
                                      sc[bq]).wait()
        pltpu.make_async_copy(src_hbm.at[brow], six[NCHUNK % NB],
                              si[NCHUNK % NB]).wait()

        plsc.subcore_barrier()

        r0 = sid * RPT
        pltpu.sync_copy(agg_sh.at[pl.ds(r0, RPT)],
                        agg_out.at[cid, pl.ds(r0, RPT)])

        @pl.when(is_last)
        def _():
            pltpu.sync_copy(agg_sh.at[pl.ds(NS * RPT, TAIL)],
                            agg_out.at[cid, pl.ds(NS * RPT, TAIL)])
        if with_counts:
            pltpu.sync_copy(cnt_sh.at[pl.ds(r0, RPT)],
                            cnt_out.at[cid, pl.ds(r0, RPT)])

            @pl.when(is_last)
            def _():
                pltpu.sync_copy(cnt_sh.at[pl.ds(NS * RPT, TAIL)],
                                cnt_out.at[cid, pl.ds(NS * RPT, TAIL)])

    return k


DA = D + CW
_sc_agg_aug = _make_sc_agg(DA, CHUNK_A, 2)
_sc_agg = _make_sc_agg(D, CHUNK_B, 2)

_RB = 1000


def _tc_body(agg_ref, cnt_ref, x_ref, wl_ref, wr_ref, b_ref, o_ref):
    agg = agg_ref[0][:, :D] + agg_ref[1][:, :D]
    cnt = cnt_ref[0, :, 0:1] + cnt_ref[1, :, 0:1]
    m = agg / jnp.maximum(cnt, 1.0)
    h = lax.dot_general(m, wl_ref[...], (((1,), (1,)), ((), ())),
                        preferred_element_type=jnp.float32)
    h += lax.dot_general(x_ref[...], wr_ref[...], (((1,), (1,)), ((), ())),
                         preferred_element_type=jnp.float32)
    h += b_ref[...]
    o_ref[...] = jnp.maximum(h, 0.0)


def _tc_body_final(agg_ref, cnt_ref, x_ref, wl_ref, wr_ref, b_ref,
                   wlin_ref, blin_ref, o_ref):
    agg = agg_ref[0][:, :D] + agg_ref[1][:, :D]
    cnt = cnt_ref[0, :, 0:1] + cnt_ref[1, :, 0:1]
    m = agg / jnp.maximum(cnt, 1.0)
    h = lax.dot_general(m, wl_ref[...], (((1,), (1,)), ((), ())),
                        preferred_element_type=jnp.float32)
    h += lax.dot_general(x_ref[...], wr_ref[...], (((1,), (1,)), ((), ())),
                         preferred_element_type=jnp.float32)
    h += b_ref[...]
    h = jnp.maximum(h, 0.0)
    o_ref[...] = lax.dot_general(h, wlin_ref[...], (((1,), (1,)), ((), ())),
                                 preferred_element_type=jnp.float32) + blin_ref[...]


def _tc_layer(aggp, cntp, x, Wl, Wr, b, Wlin=None, blin=None):
    final = Wlin is not None
    aw = aggp.shape[2]
    in_specs = [
        pl.BlockSpec((NC, _RB, aw), lambda i: (0, i, 0)),
        pl.BlockSpec((NC, _RB, CW), lambda i: (0, i, 0)),
        pl.BlockSpec((_RB, D), lambda i: (i, 0)),
        pl.BlockSpec((D, D), lambda i: (0, 0)),
        pl.BlockSpec((D, D), lambda i: (0, 0)),
        pl.BlockSpec((1, D), lambda i: (0, 0)),
    ]
    args = [aggp, cntp, x, Wl, Wr, b.reshape(1, D)]
    if final:
        in_specs += [pl.BlockSpec((D, D), lambda i: (0, 0)),
                     pl.BlockSpec((1, D), lambda i: (0, 0))]
        args += [Wlin, blin.reshape(1, D)]
    return pl.pallas_call(
        _tc_body_final if final else _tc_body,
        grid=(N // _RB,),
        in_specs=in_specs,
        out_specs=pl.BlockSpec((_RB, D), lambda i: (i, 0)),
        out_shape=jax.ShapeDtypeStruct((N, D), jnp.float32),
    )(*args)


def kernel(x, edge_index, W1l, W1r, b1, W2l, W2r, b2, W3l, W3r, b3,
           Wlin, blin):
    src_a = edge_index[0].reshape(E // CHUNK_A, CHUNK_A)
    dst_a = edge_index[1].reshape(E // CHUNK_A, CHUNK_A)
    src_b = edge_index[0].reshape(E // CHUNK_B, CHUNK_B)
    dst_b = edge_index[1].reshape(E // CHUNK_B, CHUNK_B)
    x_aug = jnp.concatenate(
        [x, jnp.ones((N, 1), jnp.float32), jnp.zeros((N, CW - 1), jnp.float32)],
        axis=1)
    (aggc,) = _sc_agg_aug(x_aug, src_a, dst_a)
    cntp = aggc[:, :, D:D + CW]
    h1 = _tc_layer(aggc, cntp, x, W1l, W1r, b1)
    (aggp2,) = _sc_agg(h1, src_b, dst_b)
    h2 = _tc_layer(aggp2, cntp, h1, W2l, W2r, b2)
    (aggp3,) = _sc_agg(h2, src_b, dst_b)
    out = _tc_layer(aggp3, cntp, h2, W3l, W3r, b3, Wlin, blin)
    return out


# speedup vs baseline: 1.0877x; 1.0877x over previous
import functools

import jax
import jax.numpy as jnp
from jax import lax
from jax.experimental import pallas as pl
from jax.experimental.pallas import tpu as pltpu
from jax.experimental.pallas import tpu_sc as plsc

N = 10000
E = 320000
D = 128
NC = 2
NS = 16
NW = NC * NS
EPW = E // NW
CHUNK_A = 80
CHUNK_B = 125
RPT = 624
TAIL = N - RPT * NS
ZROWS = 24
CW = 16


def _fill2d(ref, nrows, ncols, value):
    vec = jnp.full((16,), value, jnp.float32)

    def row(r, _):
        def col(c, _2):
            ref[r, pl.ds(c * 16, 16)] = vec
            return 0
        return lax.fori_loop(0, ncols // 16, col, 0)
    lax.fori_loop(0, nrows, row, 0)


def _make_sc_agg(with_counts, CHUNK, NB):
    NCHUNK = EPW // CHUNK
    mesh = plsc.VectorSubcoreMesh(core_axis_name="c", subcore_axis_name="s")
    out_type = [jax.ShapeDtypeStruct((NC, N, D), jnp.float32)]
    scratch = [
        pltpu.VMEM_SHARED((N, D), jnp.float32),
        pltpu.VMEM((NCHUNK, CHUNK), jnp.int32),
    ]
    scratch += [pltpu.VMEM((CHUNK,), jnp.int32) for _ in range(NB)]
    scratch += [pltpu.VMEM((CHUNK, D), jnp.float32) for _ in range(NB)]
    scratch += [pltpu.SemaphoreType.DMA for _ in range(3 * NB)]
    if with_counts:
        out_type.append(jax.ShapeDtypeStruct((NC, N, CW), jnp.float32))
        scratch += [
            pltpu.VMEM_SHARED((N, CW), jnp.float32),
            pltpu.VMEM((CHUNK, CW), jnp.float32),
        ]
        scratch += [pltpu.SemaphoreType.DMA for _ in range(NB)]

    @functools.partial(pl.kernel, mesh=mesh, out_type=tuple(out_type),
                       scratch_types=tuple(scratch),
                       compiler_params=pltpu.CompilerParams(
                           use_tc_tiling_on_sc=False))
    def k(x_hbm, src_hbm, dst_hbm, *refs):
        it = iter(refs)
        agg_out = next(it)
        cnt_out = next(it) if with_counts else None
        agg_sh = next(it)
        didx = next(it)
        six = tuple(next(it) for _ in range(NB))
        rows = tuple(next(it) for _ in range(NB))
        sg = tuple(next(it) for _ in range(NB))
        ss = tuple(next(it) for _ in range(NB))
        si = tuple(next(it) for _ in range(NB))
        if with_counts:
            cnt_sh = next(it)
            ones = next(it)
            sc = tuple(next(it) for _ in range(NB))
        cid = lax.axis_index("c")
        sid = lax.axis_index("s")
        wid = cid * NS + sid
        is_last = sid == NS - 1
        brow = wid * NCHUNK

        pltpu.async_copy(dst_hbm.at[pl.ds(brow, NCHUNK)], didx, si[0])
        pltpu.sync_copy(src_hbm.at[brow], six[0])
        pltpu.async_copy(x_hbm.at[six[0]], rows[0], sg[0])
        pltpu.async_copy(src_hbm.at[brow + 1], six[1], si[1])

        _fill2d(rows[1], CHUNK, D, 0.0)
        for j in range(RPT // CHUNK):
            pltpu.sync_copy(rows[1], agg_sh.at[pl.ds(sid * RPT + j * CHUNK, CHUNK)])
        rem = RPT - (RPT // CHUNK) * CHUNK
        pltpu.sync_copy(rows[1].at[pl.ds(0, rem)],
                        agg_sh.at[pl.ds(sid * RPT + RPT - rem, rem)])

        @pl.when(is_last)
        def _():
            pltpu.sync_copy(rows[1].at[pl.ds(0, TAIL)],
                            agg_sh.at[pl.ds(NS * RPT, TAIL)])
        if with_counts:
            _fill2d(ones, CHUNK, CW, 0.0)
            for j in range(RPT // CHUNK):
                pltpu.sync_copy(ones, cnt_sh.at[pl.ds(sid * RPT + j * CHUNK, CHUNK)])
            pltpu.sync_copy(ones.at[pl.ds(0, rem)],
                            cnt_sh.at[pl.ds(sid * RPT + RPT - rem, rem)])

            @pl.when(is_last)
            def _():
                pltpu.sync_copy(ones.at[pl.ds(0, TAIL)],
                                cnt_sh.at[pl.ds(NS * RPT, TAIL)])
            _fill2d(ones, CHUNK, CW, 1.0)
        pltpu.make_async_copy(dst_hbm.at[pl.ds(brow, NCHUNK)], didx,
                              si[0]).wait()
        plsc.subcore_barrier()

        def chunk(i, _):
            def piece(b):
                b1 = (b + 1) % NB
                b2 = (b + 2) % NB
                pltpu.make_async_copy(x_hbm.at[six[b]], rows[b], sg[b]).wait()
                nxt = jnp.minimum(i + 2, NCHUNK - 1)
                pltpu.async_copy(src_hbm.at[brow + nxt], six[b2], si[b2])
                pltpu.async_copy(rows[b], agg_sh.at[didx.at[i]], ss[b],
                                 add=True)
                if with_counts:
                    pltpu.async_copy(ones, cnt_sh.at[didx.at[i]], sc[b],
                                     add=True)

                @pl.when(i >= NB - 1)
                def _():
                    pltpu.make_async_copy(rows[b1], agg_sh.at[didx.at[i]],
                                          ss[b1]).wait()
                    if with_counts:
                        pltpu.make_async_copy(ones, cnt_sh.at[didx.at[i]],
                                              sc[b1]).wait()
                pltpu.make_async_copy(src_hbm.at[brow], six[b1], si[b1]).wait()
                pltpu.async_copy(x_hbm.at[six[b1]], rows[b1], sg[b1])

            lax.switch(lax.rem(i, NB),
                       [functools.partial(piece, b) for b in range(NB)])
            return 0
        lax.fori_loop(0, NCHUNK - 1, chunk, 0)

        lbl = (NCHUNK - 1) % NB
        pltpu.make_async_copy(x_hbm.at[six[lbl]], rows[lbl], sg[lbl]).wait()
        pltpu.async_copy(rows[lbl], agg_sh.at[didx.at[NCHUNK - 1]], ss[lbl],
                         add=True)
        if with_counts:
            pltpu.async_copy(ones, cnt_sh.at[didx.at[NCHUNK - 1]], sc[lbl],
                             add=True)
        for kq in range(NB):
            bq = (NCHUNK - 1 - kq) % NB
            pltpu.make_async_copy(rows[bq], agg_sh.at[didx.at[0]],
                                  ss[bq]).wait()
            if with_counts:
                pltpu.make_async_copy(ones, cnt_sh.at[didx.at[0]],
                                      sc[bq]).wait()
        pltpu.make_async_copy(src_hbm.at[brow], six[NCHUNK % NB],
                              si[NCHUNK % NB]).wait()

        plsc.subcore_barrier()

        r0 = sid * RPT
        pltpu.sync_copy(agg_sh.at[pl.ds(r0, RPT)],
                        agg_out.at[cid, pl.ds(r0, RPT)])

        @pl.when(is_last)
        def _():
            pltpu.sync_copy(agg_sh.at[pl.ds(NS * RPT, TAIL)],
                            agg_out.at[cid, pl.ds(NS * RPT, TAIL)])
        if with_counts:
            pltpu.sync_copy(cnt_sh.at[pl.ds(r0, RPT)],
                            cnt_out.at[cid, pl.ds(r0, RPT)])

            @pl.when(is_last)
            def _():
                pltpu.sync_copy(cnt_sh.at[pl.ds(NS * RPT, TAIL)],
                                cnt_out.at[cid, pl.ds(NS * RPT, TAIL)])

    return k


_sc_agg_cnt = _make_sc_agg(True, CHUNK_A, 2)
_sc_agg = _make_sc_agg(False, CHUNK_B, 2)

_RB = 1000


def _tc_body(agg_ref, cnt_ref, x_ref, wl_ref, wr_ref, b_ref, o_ref):
    agg = agg_ref[0] + agg_ref[1]
    cnt = cnt_ref[0, :, 0:1] + cnt_ref[1, :, 0:1]
    m = agg / jnp.maximum(cnt, 1.0)
    h = lax.dot_general(m, wl_ref[...], (((1,), (1,)), ((), ())),
                        preferred_element_type=jnp.float32)
    h += lax.dot_general(x_ref[...], wr_ref[...], (((1,), (1,)), ((), ())),
                         preferred_element_type=jnp.float32)
    h += b_ref[...]
    o_ref[...] = jnp.maximum(h, 0.0)


def _tc_body_final(agg_ref, cnt_ref, x_ref, wl_ref, wr_ref, b_ref,
                   wlin_ref, blin_ref, o_ref):
    agg = agg_ref[0] + agg_ref[1]
    cnt = cnt_ref[0, :, 0:1] + cnt_ref[1, :, 0:1]
    m = agg / jnp.maximum(cnt, 1.0)
    h = lax.dot_general(m, wl_ref[...], (((1,), (1,)), ((), ())),
                        preferred_element_type=jnp.float32)
    h += lax.dot_general(x_ref[...], wr_ref[...], (((1,), (1,)), ((), ())),
                         preferred_element_type=jnp.float32)
    h += b_ref[...]
    h = jnp.maximum(h, 0.0)
    o_ref[...] = lax.dot_general(h, wlin_ref[...], (((1,), (1,)), ((), ())),
                                 preferred_element_type=jnp.float32) + blin_ref[...]


def _tc_layer(aggp, cntp, x, Wl, Wr, b, Wlin=None, blin=None):
    final = Wlin is not None
    in_specs = [
        pl.BlockSpec((NC, _RB, D), lambda i: (0, i, 0)),
        pl.BlockSpec((NC, _RB, CW), lambda i: (0, i, 0)),
        pl.BlockSpec((_RB, D), lambda i: (i, 0)),
        pl.BlockSpec((D, D), lambda i: (0, 0)),
        pl.BlockSpec((D, D), lambda i: (0, 0)),
        pl.BlockSpec((1, D), lambda i: (0, 0)),
    ]
    args = [aggp, cntp, x, Wl, Wr, b.reshape(1, D)]
    if final:
        in_specs += [pl.BlockSpec((D, D), lambda i: (0, 0)),
                     pl.BlockSpec((1, D), lambda i: (0, 0))]
        args += [Wlin, blin.reshape(1, D)]
    return pl.pallas_call(
        _tc_body_final if final else _tc_body,
        grid=(N // _RB,),
        in_specs=in_specs,
        out_specs=pl.BlockSpec((_RB, D), lambda i: (i, 0)),
        out_shape=jax.ShapeDtypeStruct((N, D), jnp.float32),
    )(*args)


def kernel(x, edge_index, W1l, W1r, b1, W2l, W2r, b2, W3l, W3r, b3,
           Wlin, blin):
    src_a = edge_index[0].reshape(E // CHUNK_A, CHUNK_A)
    dst_a = edge_index[1].reshape(E // CHUNK_A, CHUNK_A)
    src_b = edge_index[0].reshape(E // CHUNK_B, CHUNK_B)
    dst_b = edge_index[1].reshape(E // CHUNK_B, CHUNK_B)
    aggp, cntp = _sc_agg_cnt(x, src_a, dst_a)
    h1 = _tc_layer(aggp, cntp, x, W1l, W1r, b1)
    (aggp2,) = _sc_agg(h1, src_b, dst_b)
    h2 = _tc_layer(aggp2, cntp, h1, W2l, W2r, b2)
    (aggp3,) = _sc_agg(h2, src_b, dst_b)
    out = _tc_layer(aggp3, cntp, h2, W3l, W3r, b3, Wlin, blin)
    return out
